# Initial kernel scaffold; baseline (speedup 1.0000x reference)
import jax
import jax.numpy as jnp
from jax import lax
from jax.experimental import pallas as pl
from jax.experimental.pallas import tpu as pltpu
from jax.experimental.pallas import tpu_sc as plsc

B, L = 4096, 200
N = B * L
OUT_D = 96

_LAYOUT = [
    ("bio", 3, 16),
    ("pos", 19, 16),
    ("ner", 19, 16),
    ("ans", 3, 8),
    ("clue", 2, 8),
    ("acr", 2, 8),
    ("acap", 2, 8),
    ("cap", 2, 8),
    ("pnum", 11, 8),
]

_TBL_BASE = []
_off = 0
for _n, _v, _d in _LAYOUT:
    _TBL_BASE.append(_off)
    _off += -(-(_v * _d) // 8) * 8
_TBL_WORDS = _off

_COL_BASE = []
_c = 0
for _n, _v, _d in _LAYOUT:
    _COL_BASE.append(_c)
    _c += _d

NW = 32
TOK_PER_W = N // NW
CHUNK = 512
N_CHUNKS = TOK_PER_W // CHUNK
GROUPS = CHUNK // 16


def _body(*refs):
    tag_hbm = refs[0:9]
    w_hbm = refs[9:18]
    out_hbm = refs[18]
    tbl_v = refs[19]
    tags_v = refs[20]
    out_v = refs[21]

    wid = lax.axis_index("s") * 2 + lax.axis_index("c")
    base_tok = wid * TOK_PER_W

    for t, (_n, v, d) in enumerate(_LAYOUT):
        pltpu.sync_copy(w_hbm[t], tbl_v.at[pl.ds(_TBL_BASE[t], v * d)])

    lane = lax.iota(jnp.int32, 16)
    sbase0 = lane * OUT_D

    def chunk_body(k, carry):
        off = base_tok + k * CHUNK
        for t in range(9):
            pltpu.sync_copy(tag_hbm[t].at[pl.ds(off, CHUNK)], tags_v.at[t])

        def group_body(i, carry2):
            sbase = sbase0 + i * (16 * OUT_D)
            for t, (_n, v, d) in enumerate(_LAYOUT):
                tag = tags_v[t, pl.ds(i * 16, 16)]
                row = tag * d + _TBL_BASE[t]
                for dd in range(d):
                    val = plsc.load_gather(tbl_v, [row + dd])
                    plsc.store_scatter(out_v, [sbase + (_COL_BASE[t] + dd)], val)
            return carry2

        lax.fori_loop(0, GROUPS, group_body, 0, unroll=False)
        pltpu.sync_copy(out_v, out_hbm.at[pl.ds(off * OUT_D, CHUNK * OUT_D)])
        return carry

    lax.fori_loop(0, N_CHUNKS, chunk_body, 0, unroll=False)


@jax.jit
def kernel(bio_tag, ner_tag, pos_tag, ans_tag, clue_tag, acr_tag, acap_tag,
           cap_tag, pnum_tag, bio_w, ner_w, pos_w, ans_w, clue_w, acr_w,
           acap_w, cap_w, pnum_w):
    tags = {"bio": bio_tag, "ner": ner_tag, "pos": pos_tag, "ans": ans_tag,
            "clue": clue_tag, "acr": acr_tag, "acap": acap_tag, "cap": cap_tag,
            "pnum": pnum_tag}
    ws = {"bio": bio_w, "ner": ner_w, "pos": pos_w, "ans": ans_w,
          "clue": clue_w, "acr": acr_w, "acap": acap_w, "cap": cap_w,
          "pnum": pnum_w}
    tag_flat = [tags[n].reshape(N).astype(jnp.int32) for n, _v, _d in _LAYOUT]
    w_flat = [ws[n].reshape(-1).astype(jnp.float32) for n, _v, _d in _LAYOUT]

    mesh = plsc.VectorSubcoreMesh(core_axis_name="c", subcore_axis_name="s")
    run = pl.kernel(
        _body,
        out_type=jax.ShapeDtypeStruct((N * OUT_D,), jnp.float32),
        mesh=mesh,
        scratch_types=[
            pltpu.VMEM((_TBL_WORDS,), jnp.float32),
            pltpu.VMEM((9, CHUNK), jnp.int32),
            pltpu.VMEM((CHUNK * OUT_D,), jnp.float32),
        ],
    )
    out = run(*tag_flat, *w_flat)
    return out.reshape(B, L, OUT_D)


# speedup vs baseline: 7.1961x; 7.1961x over previous
import jax
import jax.numpy as jnp
from jax import lax
from jax.experimental import pallas as pl
from jax.experimental.pallas import tpu as pltpu
from jax.experimental.pallas import tpu_sc as plsc

B, L = 4096, 200
N = B * L
OUT_D = 96

_LAYOUT = [
    ("bio", 3, 16),
    ("pos", 19, 16),
    ("ner", 19, 16),
    ("ans", 3, 8),
    ("clue", 2, 8),
    ("acr", 2, 8),
    ("acap", 2, 8),
    ("cap", 2, 8),
    ("pnum", 11, 8),
]

_TBL_BASE = []
_off = 0
for _n, _v, _d in _LAYOUT:
    _TBL_BASE.append(_off)
    _off += -(-(_v * _d) // 8) * 8
_TBL_WORDS = _off

_COL_BASE = []
_c = 0
for _n, _v, _d in _LAYOUT:
    _COL_BASE.append(_c)
    _c += _d

NW = 32
TOK_PER_W = N // NW
CHUNK = 512
N_CHUNKS = TOK_PER_W // CHUNK
GROUPS = CHUNK // 16


def _body(*refs):
    tag_hbm = refs[0:9]
    w_hbm = refs[9:18]
    out_hbm = refs[18]
    tbl_v = refs[19]
    tags_v = refs[20:29]
    out_v = refs[29]

    wid = lax.axis_index("s") * 2 + lax.axis_index("c")
    base_tok = wid * TOK_PER_W

    for t, (_n, v, d) in enumerate(_LAYOUT):
        pltpu.sync_copy(w_hbm[t], tbl_v.at[pl.ds(_TBL_BASE[t], v * d)])

    lane = lax.iota(jnp.int32, 16)
    sbase0 = lane * OUT_D

    def chunk_body(k, carry):
        off = base_tok + k * CHUNK
        for t in range(9):
            pltpu.sync_copy(tag_hbm[t].at[pl.ds(off, CHUNK)], tags_v[t])

        def group_body(i, carry2):
            sbase = sbase0 + i * (16 * OUT_D)
            for t, (_n, v, d) in enumerate(_LAYOUT):
                tag = tags_v[t][pl.ds(i * 16, 16)]
                row = tag * d + _TBL_BASE[t]
                for dd in range(d):
                    val = plsc.load_gather(tbl_v, [row + dd])
                    plsc.store_scatter(out_v, [sbase + (_COL_BASE[t] + dd)], val)
            return carry2

        lax.fori_loop(0, GROUPS, group_body, 0, unroll=False)
        pltpu.sync_copy(out_v, out_hbm.at[pl.ds(off * OUT_D, CHUNK * OUT_D)])
        return carry

    lax.fori_loop(0, N_CHUNKS, chunk_body, 0, unroll=False)


@jax.jit
def kernel(bio_tag, ner_tag, pos_tag, ans_tag, clue_tag, acr_tag, acap_tag,
           cap_tag, pnum_tag, bio_w, ner_w, pos_w, ans_w, clue_w, acr_w,
           acap_w, cap_w, pnum_w):
    tags = {"bio": bio_tag, "ner": ner_tag, "pos": pos_tag, "ans": ans_tag,
            "clue": clue_tag, "acr": acr_tag, "acap": acap_tag, "cap": cap_tag,
            "pnum": pnum_tag}
    ws = {"bio": bio_w, "ner": ner_w, "pos": pos_w, "ans": ans_w,
          "clue": clue_w, "acr": acr_w, "acap": acap_w, "cap": cap_w,
          "pnum": pnum_w}
    tag_flat = [tags[n].reshape(N).astype(jnp.int32) for n, _v, _d in _LAYOUT]
    w_flat = [ws[n].reshape(-1).astype(jnp.float32) for n, _v, _d in _LAYOUT]

    mesh = plsc.VectorSubcoreMesh(core_axis_name="c", subcore_axis_name="s")
    run = pl.kernel(
        _body,
        out_type=jax.ShapeDtypeStruct((N * OUT_D,), jnp.float32),
        mesh=mesh,
        compiler_params=pltpu.CompilerParams(needs_layout_passes=False),
        scratch_types=[
            pltpu.VMEM((_TBL_WORDS,), jnp.float32),
            *[pltpu.VMEM((CHUNK,), jnp.int32) for _ in range(9)],
            pltpu.VMEM((CHUNK * OUT_D,), jnp.float32),
        ],
    )
    out = run(*tag_flat, *w_flat)
    return out.reshape(B, L, OUT_D)


# speedup vs baseline: 17.8717x; 2.4835x over previous
import jax
import jax.numpy as jnp
from jax import lax
from jax.experimental import pallas as pl
from jax.experimental.pallas import tpu as pltpu
from jax.experimental.pallas import tpu_sc as plsc

B, L = 4096, 200
N = B * L
OUT_D = 96

_SRC = [
    ("bio", 3, 16),
    ("pos", 19, 16),
    ("ner", 19, 16),
    ("ans", 3, 8),
    ("clue", 2, 8),
    ("acr", 2, 8),
    ("acap", 2, 8),
    ("cap", 2, 8),
    ("pnum", 11, 8),
]

_MERGED = [
    (0, 3, 0, None),
    (16, 19, 1, None),
    (32, 19, 2, None),
    (48, 6, 3, 4),
    (64, 4, 5, 6),
    (80, 22, 7, 8),
]

NW = 32
TOK_PER_W = N // NW
CHUNK = 512
N_CHUNKS = TOK_PER_W // CHUNK
GROUPS = CHUNK // 16


def _body(*refs):
    tag_hbm = refs[0:9]
    w_hbm = refs[9:18]
    out_hbm = refs[18]
    tbl_sp = refs[19:25]
    tags_v = refs[25:34]
    pidx_v = refs[34:37]
    row_v = refs[37:43]
    sem = refs[43]

    cid = lax.axis_index("c")
    sid = lax.axis_index("s")
    wid = sid * 2 + cid
    base_tok = wid * TOK_PER_W

    @pl.when(sid == 0)
    def _stage():
        for m, (_cb, rows, a, b) in enumerate(_MERGED):
            if b is None:
                pltpu.sync_copy(w_hbm[a], tbl_sp[m])
            else:
                da = _SRC[a][2]
                vb = _SRC[b][1]
                for ra in range(_SRC[a][1]):
                    for rb in range(vb):
                        r = ra * vb + rb
                        pltpu.sync_copy(
                            w_hbm[a].at[pl.ds(ra * da, 8)],
                            tbl_sp[m].at[r, pl.ds(0, 8)])
                        pltpu.sync_copy(
                            w_hbm[b].at[pl.ds(rb * 8, 8)],
                            tbl_sp[m].at[r, pl.ds(8, 8)])

    plsc.subcore_barrier()

    def chunk_body(k, carry):
        off = base_tok + k * CHUNK
        for t in range(9):
            pltpu.sync_copy(tag_hbm[t].at[pl.ds(off, CHUNK)], tags_v[t])

        def group_body(g, carry2):
            s = pl.ds(g * 16, 16)
            for p, (_cb, _rows, a, b) in enumerate(_MERGED[3:]):
                pidx_v[p][s] = tags_v[a][s] * _SRC[b][1] + tags_v[b][s]
            return carry2

        lax.fori_loop(0, GROUPS, group_body, 0, unroll=False)

        idx_refs = [tags_v[0], tags_v[1], tags_v[2],
                    pidx_v[0], pidx_v[1], pidx_v[2]]
        cps = []
        for m in range(6):
            cps.append(pltpu.async_copy(tbl_sp[m].at[idx_refs[m]],
                                        row_v[m], sem))
        for m in range(6):
            cps[m].wait()
        for m, (cb, _rows, _a, _b) in enumerate(_MERGED):
            pltpu.sync_copy(row_v[m],
                            out_hbm.at[pl.ds(off, CHUNK), pl.ds(cb, 16)])
        return carry

    lax.fori_loop(0, N_CHUNKS, chunk_body, 0, unroll=False)


@jax.jit
def kernel(bio_tag, ner_tag, pos_tag, ans_tag, clue_tag, acr_tag, acap_tag,
           cap_tag, pnum_tag, bio_w, ner_w, pos_w, ans_w, clue_w, acr_w,
           acap_w, cap_w, pnum_w):
    tags = {"bio": bio_tag, "ner": ner_tag, "pos": pos_tag, "ans": ans_tag,
            "clue": clue_tag, "acr": acr_tag, "acap": acap_tag, "cap": cap_tag,
            "pnum": pnum_tag}
    ws = {"bio": bio_w, "ner": ner_w, "pos": pos_w, "ans": ans_w,
          "clue": clue_w, "acr": acr_w, "acap": acap_w, "cap": cap_w,
          "pnum": pnum_w}
    tag_flat = [tags[n].reshape(N).astype(jnp.int32) for n, _v, _d in _SRC]
    w_flat = [ws[n].astype(jnp.float32) if d == 16
              else ws[n].reshape(-1).astype(jnp.float32)
              for n, _v, d in _SRC]

    mesh = plsc.VectorSubcoreMesh(core_axis_name="c", subcore_axis_name="s")
    run = pl.kernel(
        _body,
        out_type=jax.ShapeDtypeStruct((N, OUT_D), jnp.float32),
        mesh=mesh,
        compiler_params=pltpu.CompilerParams(needs_layout_passes=False,
                                             use_tc_tiling_on_sc=False),
        scratch_types=[
            *[pltpu.VMEM_SHARED((rows, 16), jnp.float32)
              for _cb, rows, _a, _b in _MERGED],
            *[pltpu.VMEM((CHUNK,), jnp.int32) for _ in range(9)],
            *[pltpu.VMEM((CHUNK,), jnp.int32) for _ in range(3)],
            *[pltpu.VMEM((CHUNK, 16), jnp.float32) for _ in range(6)],
            pltpu.SemaphoreType.DMA,
        ],
    )
    out = run(*tag_flat, *w_flat)
    return out.reshape(B, L, OUT_D)


# speedup vs baseline: 23.3371x; 1.3058x over previous
import jax
import jax.numpy as jnp
from jax import lax
from jax.experimental import pallas as pl
from jax.experimental.pallas import tpu as pltpu
from jax.experimental.pallas import tpu_sc as plsc

B, L = 4096, 200
N = B * L
OUT_D = 96

_SRC = [
    ("bio", 3, 16),
    ("pos", 19, 16),
    ("ner", 19, 16),
    ("ans", 3, 8),
    ("clue", 2, 8),
    ("acr", 2, 8),
    ("acap", 2, 8),
    ("cap", 2, 8),
    ("pnum", 11, 8),
]

_MERGED = [
    (0, 3, 0, None),
    (16, 19, 1, None),
    (32, 19, 2, None),
    (48, 6, 3, 4),
    (64, 4, 5, 6),
    (80, 22, 7, 8),
]

NW = 32
TOK_PER_W = N // NW
CHUNK = 512
N_CHUNKS = TOK_PER_W // CHUNK
GROUPS = CHUNK // 16


def _body(*refs):
    tag_hbm = refs[0:9]
    w_hbm = refs[9:18]
    out_hbm = refs[18]
    tbl_sp = refs[19:25]
    tags_v = [refs[25:34], refs[34:43]]
    pidx_v = [refs[43:46], refs[46:49]]
    row_v = [refs[49:55], refs[55:61]]
    sem_t = refs[61:63]
    sem_g = refs[63:65]
    sem_w = refs[65:67]

    cid = lax.axis_index("c")
    sid = lax.axis_index("s")
    wid = sid * 2 + cid
    base_tok = wid * TOK_PER_W

    @pl.when(sid == 0)
    def _stage():
        for m, (_cb, rows, a, b) in enumerate(_MERGED):
            if b is None:
                pltpu.sync_copy(w_hbm[a], tbl_sp[m])
            else:
                da = _SRC[a][2]
                vb = _SRC[b][1]
                for ra in range(_SRC[a][1]):
                    for rb in range(vb):
                        r = ra * vb + rb
                        pltpu.sync_copy(
                            w_hbm[a].at[pl.ds(ra * da, 8)],
                            tbl_sp[m].at[r, pl.ds(0, 8)])
                        pltpu.sync_copy(
                            w_hbm[b].at[pl.ds(rb * 8, 8)],
                            tbl_sp[m].at[r, pl.ds(8, 8)])

    plsc.subcore_barrier()

    idx_refs = [[tags_v[p][0], tags_v[p][1], tags_v[p][2],
                 pidx_v[p][0], pidx_v[p][1], pidx_v[p][2]] for p in (0, 1)]

    def issue_tags(p, k):
        off = base_tok + k * CHUNK
        for t in range(9):
            pltpu.async_copy(tag_hbm[t].at[pl.ds(off, CHUNK)],
                             tags_v[p][t], sem_t[p])

    def wait_tags(p):
        for t in range(9):
            pltpu.make_async_copy(tag_hbm[t].at[pl.ds(0, CHUNK)],
                                  tags_v[p][t], sem_t[p]).wait()

    def compute_pidx(p):
        def group_body(g, carry):
            s = pl.ds(g * 16, 16)
            for q, (_cb, _rows, a, b) in enumerate(_MERGED[3:]):
                pidx_v[p][q][s] = tags_v[p][a][s] * _SRC[b][1] + tags_v[p][b][s]
            return carry
        lax.fori_loop(0, GROUPS, group_body, 0, unroll=False)

    def issue_gathers(p):
        for m in range(6):
            pltpu.async_copy(tbl_sp[m].at[idx_refs[p][m]], row_v[p][m],
                             sem_g[p])

    def wait_gathers(p):
        for m in range(6):
            pltpu.make_async_copy(tbl_sp[m].at[idx_refs[p][m]], row_v[p][m],
                                  sem_g[p]).wait()

    def issue_writes(p, k):
        off = base_tok + k * CHUNK
        for m, (cb, _rows, _a, _b) in enumerate(_MERGED):
            pltpu.async_copy(row_v[p][m],
                             out_hbm.at[pl.ds(off, CHUNK), pl.ds(cb, 16)],
                             sem_w[p])

    def wait_writes(p):
        for m, (cb, _rows, _a, _b) in enumerate(_MERGED):
            pltpu.make_async_copy(row_v[p][m],
                                  out_hbm.at[pl.ds(0, CHUNK), pl.ds(cb, 16)],
                                  sem_w[p]).wait()

    issue_tags(0, 0)

    def pair_body(j, carry):
        for p in (0, 1):
            k = j * 2 + p
            wait_tags(p)
            compute_pidx(p)

            @pl.when(j >= 1)
            def _drain_writes():
                wait_writes(p)

            issue_gathers(p)

            @pl.when((j >= 1) | (p == 1))
            def _flush_prev():
                wait_gathers(1 - p)
                issue_writes(1 - p, k - 1)

            if p == 0:
                issue_tags(1, k + 1)
            else:
                @pl.when(j < N_CHUNKS // 2 - 1)
                def _prefetch():
                    issue_tags(0, k + 1)
        return carry

    lax.fori_loop(0, N_CHUNKS // 2, pair_body, 0, unroll=False)

    wait_gathers(1)
    issue_writes(1, N_CHUNKS - 1)
    wait_writes(0)
    wait_writes(1)


@jax.jit
def kernel(bio_tag, ner_tag, pos_tag, ans_tag, clue_tag, acr_tag, acap_tag,
           cap_tag, pnum_tag, bio_w, ner_w, pos_w, ans_w, clue_w, acr_w,
           acap_w, cap_w, pnum_w):
    tags = {"bio": bio_tag, "ner": ner_tag, "pos": pos_tag, "ans": ans_tag,
            "clue": clue_tag, "acr": acr_tag, "acap": acap_tag, "cap": cap_tag,
            "pnum": pnum_tag}
    ws = {"bio": bio_w, "ner": ner_w, "pos": pos_w, "ans": ans_w,
          "clue": clue_w, "acr": acr_w, "acap": acap_w, "cap": cap_w,
          "pnum": pnum_w}
    tag_flat = [tags[n].reshape(N).astype(jnp.int32) for n, _v, _d in _SRC]
    w_flat = [ws[n].astype(jnp.float32) if d == 16
              else ws[n].reshape(-1).astype(jnp.float32)
              for n, _v, d in _SRC]

    mesh = plsc.VectorSubcoreMesh(core_axis_name="c", subcore_axis_name="s")
    run = pl.kernel(
        _body,
        out_type=jax.ShapeDtypeStruct((N, OUT_D), jnp.float32),
        mesh=mesh,
        compiler_params=pltpu.CompilerParams(needs_layout_passes=False,
                                             use_tc_tiling_on_sc=False),
        scratch_types=[
            *[pltpu.VMEM_SHARED((rows, 16), jnp.float32)
              for _cb, rows, _a, _b in _MERGED],
            *[pltpu.VMEM((CHUNK,), jnp.int32) for _ in range(18)],
            *[pltpu.VMEM((CHUNK,), jnp.int32) for _ in range(6)],
            *[pltpu.VMEM((CHUNK, 16), jnp.float32) for _ in range(12)],
            *[pltpu.SemaphoreType.DMA for _ in range(6)],
        ],
    )
    out = run(*tag_flat, *w_flat)
    return out.reshape(B, L, OUT_D)
